# Initial kernel scaffold; baseline (speedup 1.0000x reference)
import jax
import jax.numpy as jnp
from jax.experimental import pallas as pl


def kernel(B_z, G_z, x_b_batch, x_g_batch, W, b):
    raise NotImplementedError("write your pallas kernel here")


# speedup vs baseline: 19.3208x; 19.3208x over previous
import functools

import jax
import jax.numpy as jnp
from jax.experimental import pallas as pl
from jax.experimental.pallas import tpu as pltpu

_G = 128
_C = 256


def _pool_kernel(ib_ref, ig_ref, b_ref, g_ref, w_ref, bias_ref, out_ref,
                 accb_ref, accg_ref, *, nsteps):
    i = pl.program_id(0)

    @pl.when(i == 0)
    def _init():
        accb_ref[...] = jnp.zeros_like(accb_ref)
        accg_ref[...] = jnp.zeros_like(accg_ref)

    w1 = w_ref[0, :_C]
    w2 = w_ref[0, _C:]
    bb = b_ref[...]
    gb = g_ref[...]
    sv_b = jnp.sum(bb * w1[None, :], axis=1, keepdims=True)
    sv_g = jnp.sum(gb * w2[None, :], axis=1, keepdims=True)
    ones = jnp.ones_like(sv_b)
    svc_b = jnp.concatenate([sv_b, ones], axis=1)
    svc_g = jnp.concatenate([sv_g, ones], axis=1)
    ids_b = ib_ref[0]
    ids_g = ig_ref[0]
    seg = jax.lax.broadcasted_iota(jnp.int32, (_G, ids_b.shape[1]), 0)
    oh_b = (seg == ids_b).astype(jnp.float32)
    oh_g = (seg == ids_g).astype(jnp.float32)
    dn = (((1,), (0,)), ((), ()))
    accb_ref[...] += jax.lax.dot_general(
        oh_b, svc_b, dn, preferred_element_type=jnp.float32)
    accg_ref[...] += jax.lax.dot_general(
        oh_g, svc_g, dn, preferred_element_type=jnp.float32)

    @pl.when(i == nsteps - 1)
    def _fin():
        ab = accb_ref[...]
        ag = accg_ref[...]
        res = (ab[:, 0] / jnp.maximum(ab[:, 1], 1.0)
               + ag[:, 0] / jnp.maximum(ag[:, 1], 1.0)
               + bias_ref[0, 0])
        out_ref[...] = res[:, None]


def kernel(B_z, G_z, x_b_batch, x_g_batch, W, b):
    nb, c = B_z.shape
    r = 2000
    nsteps = nb // r
    ib = x_b_batch.astype(jnp.int32).reshape(nsteps, 1, r)
    ig = x_g_batch.astype(jnp.int32).reshape(nsteps, 1, r)
    bias = b.reshape(1, 1)
    out = pl.pallas_call(
        functools.partial(_pool_kernel, nsteps=nsteps),
        grid=(nsteps,),
        in_specs=[
            pl.BlockSpec((1, 1, r), lambda i: (i, 0, 0)),
            pl.BlockSpec((1, 1, r), lambda i: (i, 0, 0)),
            pl.BlockSpec((r, c), lambda i: (i, 0)),
            pl.BlockSpec((r, c), lambda i: (i, 0)),
            pl.BlockSpec((1, 2 * _C), lambda i: (0, 0)),
            pl.BlockSpec((1, 1), lambda i: (0, 0)),
        ],
        out_specs=pl.BlockSpec((_G, 1), lambda i: (0, 0)),
        out_shape=jax.ShapeDtypeStruct((_G, 1), jnp.float32),
        scratch_shapes=[pltpu.VMEM((_G, 2), jnp.float32),
                        pltpu.VMEM((_G, 2), jnp.float32)],
        compiler_params=pltpu.CompilerParams(
            dimension_semantics=("arbitrary",)),
    )(ib, ig, B_z, G_z, W, bias)
    return out


# speedup vs baseline: 23.5228x; 1.2175x over previous
import functools

import jax
import jax.numpy as jnp
from jax.experimental import pallas as pl
from jax.experimental.pallas import tpu as pltpu

_G = 128
_C = 256


def _pool_kernel(ib_ref, ig_ref, b_ref, g_ref, w_ref, bias_ref, out_ref,
                 accb_ref, accg_ref, *, nsteps):
    i = pl.program_id(0)

    @pl.when(i == 0)
    def _init():
        accb_ref[...] = jnp.zeros_like(accb_ref)
        accg_ref[...] = jnp.zeros_like(accg_ref)

    w1 = w_ref[0, :_C]
    w2 = w_ref[0, _C:]
    bb = b_ref[...]
    gb = g_ref[...]
    sv_b = jnp.sum(bb * w1[None, :], axis=1, keepdims=True)
    sv_g = jnp.sum(gb * w2[None, :], axis=1, keepdims=True)
    ones = jnp.ones_like(sv_b)
    svc_b = jnp.concatenate([sv_b, ones], axis=1)
    svc_g = jnp.concatenate([sv_g, ones], axis=1)
    ids_b = ib_ref[0]
    ids_g = ig_ref[0]
    seg = jax.lax.broadcasted_iota(jnp.int32, (_G, ids_b.shape[1]), 0)
    oh_b = (seg == ids_b).astype(jnp.float32)
    oh_g = (seg == ids_g).astype(jnp.float32)
    dn = (((1,), (0,)), ((), ()))
    accb_ref[...] += jax.lax.dot_general(
        oh_b, svc_b, dn, preferred_element_type=jnp.float32)
    accg_ref[...] += jax.lax.dot_general(
        oh_g, svc_g, dn, preferred_element_type=jnp.float32)

    @pl.when(i == nsteps - 1)
    def _fin():
        ab = accb_ref[...]
        ag = accg_ref[...]
        res = (ab[:, 0] / jnp.maximum(ab[:, 1], 1.0)
               + ag[:, 0] / jnp.maximum(ag[:, 1], 1.0)
               + bias_ref[0, 0])
        out_ref[...] = res[:, None]


def kernel(B_z, G_z, x_b_batch, x_g_batch, W, b):
    nb, c = B_z.shape
    r = 5000
    nsteps = nb // r
    ib = x_b_batch.astype(jnp.int32).reshape(nsteps, 1, r)
    ig = x_g_batch.astype(jnp.int32).reshape(nsteps, 1, r)
    bias = b.reshape(1, 1)
    out = pl.pallas_call(
        functools.partial(_pool_kernel, nsteps=nsteps),
        grid=(nsteps,),
        in_specs=[
            pl.BlockSpec((1, 1, r), lambda i: (i, 0, 0)),
            pl.BlockSpec((1, 1, r), lambda i: (i, 0, 0)),
            pl.BlockSpec((r, c), lambda i: (i, 0)),
            pl.BlockSpec((r, c), lambda i: (i, 0)),
            pl.BlockSpec((1, 2 * _C), lambda i: (0, 0)),
            pl.BlockSpec((1, 1), lambda i: (0, 0)),
        ],
        out_specs=pl.BlockSpec((_G, 1), lambda i: (0, 0)),
        out_shape=jax.ShapeDtypeStruct((_G, 1), jnp.float32),
        scratch_shapes=[pltpu.VMEM((_G, 2), jnp.float32),
                        pltpu.VMEM((_G, 2), jnp.float32)],
        compiler_params=pltpu.CompilerParams(
            dimension_semantics=("arbitrary",)),
    )(ib, ig, B_z, G_z, W, bias)
    return out
